# baseline (speedup 1.0000x reference)
import functools

import jax
import jax.numpy as jnp
from jax.experimental import pallas as pl


def _pick_block(n: int, cap: int = 512) -> int:
    best = 8
    for b in range(8, cap + 1, 8):
        if n % b == 0:
            best = b
    return best


def _layer1_kernel(bi, adj_ref, x_ref, w1_ref, h_ref):
    i = pl.program_id(0)
    a = adj_ref[...]
    xf = x_ref[...]
    deg = jnp.sum(a, axis=1, keepdims=True)
    neigh = jax.lax.dot_general(
        a, xf, (((1,), (0,)), ((), ())),
        preferred_element_type=jnp.float32,
        precision=jax.lax.Precision.HIGHEST,
    ) / (deg + 1.0)
    xi = x_ref[pl.ds(i * bi, bi), :]
    w1 = w1_ref[...]
    d = xf.shape[1]
    h = (
        jnp.dot(xi, w1[:, :d].T, preferred_element_type=jnp.float32,
                precision=jax.lax.Precision.HIGHEST)
        + jnp.dot(neigh, w1[:, d:].T, preferred_element_type=jnp.float32,
                  precision=jax.lax.Precision.HIGHEST)
    )
    h = jnp.maximum(h, 0.0)
    nrm = jnp.sqrt(jnp.sum(h * h, axis=1, keepdims=True))
    h = h / jnp.maximum(nrm, 1e-12)
    h_ref[...] = h


def _layer2_kernel(bi, adj_ref, h_ref, w2_ref, cw_ref, cb_ref, out_ref):
    i = pl.program_id(0)
    a = adj_ref[...]
    hf = h_ref[...]
    deg = jnp.sum(a, axis=1, keepdims=True)
    neigh = jax.lax.dot_general(
        a, hf, (((1,), (0,)), ((), ())),
        preferred_element_type=jnp.float32,
        precision=jax.lax.Precision.HIGHEST,
    ) / (deg + 1.0)
    hi = h_ref[pl.ds(i * bi, bi), :]
    w2 = w2_ref[...]
    hdim = hf.shape[1]
    z = (
        jnp.dot(hi, w2[:, :hdim].T, preferred_element_type=jnp.float32,
                precision=jax.lax.Precision.HIGHEST)
        + jnp.dot(neigh, w2[:, hdim:].T, preferred_element_type=jnp.float32,
                  precision=jax.lax.Precision.HIGHEST)
    )
    out = jnp.dot(z, cw_ref[...].T, preferred_element_type=jnp.float32,
                  precision=jax.lax.Precision.HIGHEST) + cb_ref[0:1, :]
    out_ref[...] = out


@jax.jit
def kernel(adj, x, W1, W2, clf_w, clf_b):
    n, d = x.shape
    h_dim = W1.shape[0]
    c = clf_w.shape[0]
    bi = _pick_block(n)
    grid = (n // bi,)

    h = pl.pallas_call(
        functools.partial(_layer1_kernel, bi),
        grid=grid,
        in_specs=[
            pl.BlockSpec((bi, n), lambda i: (i, 0)),
            pl.BlockSpec((n, d), lambda i: (0, 0)),
            pl.BlockSpec(W1.shape, lambda i: (0, 0)),
        ],
        out_specs=pl.BlockSpec((bi, h_dim), lambda i: (i, 0)),
        out_shape=jax.ShapeDtypeStruct((n, h_dim), jnp.float32),
    )(adj, x, W1)

    cb = jnp.broadcast_to(clf_b.reshape(1, c), (8, c))
    out = pl.pallas_call(
        functools.partial(_layer2_kernel, bi),
        grid=grid,
        in_specs=[
            pl.BlockSpec((bi, n), lambda i: (i, 0)),
            pl.BlockSpec((n, h_dim), lambda i: (0, 0)),
            pl.BlockSpec(W2.shape, lambda i: (0, 0)),
            pl.BlockSpec(clf_w.shape, lambda i: (0, 0)),
            pl.BlockSpec((8, c), lambda i: (0, 0)),
        ],
        out_specs=pl.BlockSpec((bi, c), lambda i: (i, 0)),
        out_shape=jax.ShapeDtypeStruct((n, c), jnp.float32),
    )(adj, h, W2, clf_w, cb)
    return out


# speedup vs baseline: 2.7880x; 2.7880x over previous
import functools

import jax
import jax.numpy as jnp
from jax.experimental import pallas as pl


def _pick_block(n: int, cap: int = 512) -> int:
    best = 8
    for b in range(8, cap + 1, 8):
        if n % b == 0:
            best = b
    return best


def _layer1_kernel(bi, adj_ref, x_ref, w1_ref, h_ref):
    i = pl.program_id(0)
    a = adj_ref[...]
    xf = x_ref[...]
    deg = jnp.sum(a, axis=1, keepdims=True)
    neigh = jax.lax.dot_general(
        a, xf, (((1,), (0,)), ((), ())),
        preferred_element_type=jnp.float32,
        precision=jax.lax.Precision.DEFAULT,
    ) / (deg + 1.0)
    xi = x_ref[pl.ds(i * bi, bi), :]
    w1 = w1_ref[...]
    d = xf.shape[1]
    h = (
        jnp.dot(xi, w1[:, :d].T, preferred_element_type=jnp.float32,
                precision=jax.lax.Precision.HIGHEST)
        + jnp.dot(neigh, w1[:, d:].T, preferred_element_type=jnp.float32,
                  precision=jax.lax.Precision.HIGHEST)
    )
    h = jnp.maximum(h, 0.0)
    nrm = jnp.sqrt(jnp.sum(h * h, axis=1, keepdims=True))
    h = h / jnp.maximum(nrm, 1e-12)
    h_ref[...] = h


def _layer2_kernel(bi, adj_ref, h_ref, w2_ref, cw_ref, cb_ref, out_ref):
    i = pl.program_id(0)
    a = adj_ref[...]
    hf = h_ref[...]
    deg = jnp.sum(a, axis=1, keepdims=True)
    neigh = jax.lax.dot_general(
        a, hf, (((1,), (0,)), ((), ())),
        preferred_element_type=jnp.float32,
        precision=jax.lax.Precision.DEFAULT,
    ) / (deg + 1.0)
    hi = h_ref[pl.ds(i * bi, bi), :]
    w2 = w2_ref[...]
    hdim = hf.shape[1]
    z = (
        jnp.dot(hi, w2[:, :hdim].T, preferred_element_type=jnp.float32,
                precision=jax.lax.Precision.HIGHEST)
        + jnp.dot(neigh, w2[:, hdim:].T, preferred_element_type=jnp.float32,
                  precision=jax.lax.Precision.HIGHEST)
    )
    out = jnp.dot(z, cw_ref[...].T, preferred_element_type=jnp.float32,
                  precision=jax.lax.Precision.HIGHEST) + cb_ref[0:1, :]
    out_ref[...] = out


@jax.jit
def kernel(adj, x, W1, W2, clf_w, clf_b):
    n, d = x.shape
    h_dim = W1.shape[0]
    c = clf_w.shape[0]
    bi = _pick_block(n)
    grid = (n // bi,)

    h = pl.pallas_call(
        functools.partial(_layer1_kernel, bi),
        grid=grid,
        in_specs=[
            pl.BlockSpec((bi, n), lambda i: (i, 0)),
            pl.BlockSpec((n, d), lambda i: (0, 0)),
            pl.BlockSpec(W1.shape, lambda i: (0, 0)),
        ],
        out_specs=pl.BlockSpec((bi, h_dim), lambda i: (i, 0)),
        out_shape=jax.ShapeDtypeStruct((n, h_dim), jnp.float32),
    )(adj, x, W1)

    cb = jnp.broadcast_to(clf_b.reshape(1, c), (8, c))
    out = pl.pallas_call(
        functools.partial(_layer2_kernel, bi),
        grid=grid,
        in_specs=[
            pl.BlockSpec((bi, n), lambda i: (i, 0)),
            pl.BlockSpec((n, h_dim), lambda i: (0, 0)),
            pl.BlockSpec(W2.shape, lambda i: (0, 0)),
            pl.BlockSpec(clf_w.shape, lambda i: (0, 0)),
            pl.BlockSpec((8, c), lambda i: (0, 0)),
        ],
        out_specs=pl.BlockSpec((bi, c), lambda i: (i, 0)),
        out_shape=jax.ShapeDtypeStruct((n, c), jnp.float32),
    )(adj, h, W2, clf_w, cb)
    return out
